# Initial kernel scaffold; baseline (speedup 1.0000x reference)
import functools

import jax
import jax.numpy as jnp
from jax import lax
from jax.experimental import pallas as pl
from jax.experimental.pallas import tpu as pltpu
from jax.experimental.pallas import tpu_sc as plsc

SLOPE = 0.2
L = 16
NC = 2
NS = 16
NW = NC * NS
B = 128


def _matmul_scores_kernel(x_ref, w_ref, b_ref, wi_ref, s_ref):
    wi = jnp.dot(x_ref[...], w_ref[...], preferred_element_type=jnp.float32)
    wi_ref[...] = wi
    s_ref[...] = lax.dot_general(
        b_ref[...], wi, (((0,), (1,)), ((), ())),
        preferred_element_type=jnp.float32)


def _combine_kernel(p_ref, d_ref, o_ref):
    psum = p_ref[0] + p_ref[1]
    den = d_ref[0] + d_ref[1]
    o_ref[...] = psum / den


def _make_edge_kernel(npad, d, nblk):
    rpt = npad // NS
    mesh = plsc.VectorSubcoreMesh(core_axis_name="c", subcore_axis_name="s")

    @functools.partial(
        pl.kernel,
        out_type=(
            jax.ShapeDtypeStruct((NC, npad, d), jnp.float32),
            jax.ShapeDtypeStruct((NC, npad), jnp.float32),
        ),
        mesh=mesh,
        scratch_types=[
            pltpu.VMEM((npad,), jnp.float32),
            pltpu.VMEM((npad,), jnp.float32),
            pltpu.VMEM((nblk, B), jnp.int32),
            pltpu.VMEM((nblk, B), jnp.int32),
            pltpu.VMEM((nblk, B), jnp.float32),
            pltpu.VMEM((B, d), jnp.float32),
            pltpu.VMEM((npad // NS,), jnp.float32),
            pltpu.VMEM_SHARED((npad, d), jnp.float32),
            pltpu.VMEM_SHARED((npad,), jnp.float32),
            pltpu.SemaphoreType.DMA,
        ],
    )
    def edge_kernel(wi_hbm, s1_hbm, s2_hbm, src_hbm, dst_hbm,
                    outp_hbm, denp_hbm,
                    s1v, s2v, srcv, dstv, alphav, rows, zden,
                    out_sh, den_sh, sem):
        c = lax.axis_index("c")
        s = lax.axis_index("s")
        w = s * NC + c

        cp1 = pltpu.async_copy(s1_hbm, s1v, sem)
        cp2 = pltpu.async_copy(s2_hbm, s2v, sem)
        cp3 = pltpu.async_copy(src_hbm.at[w], srcv, sem)
        cp4 = pltpu.async_copy(dst_hbm.at[w], dstv, sem)

        def zrow_body(b, carry):
            for k in range(d // L):
                rows[b, pl.ds(k * L, L)] = jnp.zeros((L,), jnp.float32)
            return carry
        lax.fori_loop(0, B, zrow_body, 0)
        for k in range(rpt // L):
            zden[pl.ds(k * L, L)] = jnp.zeros((L,), jnp.float32)

        r0 = s * rpt
        for k in range(rpt // B):
            pltpu.sync_copy(rows, out_sh.at[pl.ds(r0 + k * B, B)])
        pltpu.sync_copy(zden, den_sh.at[pl.ds(r0, rpt)])

        cp1.wait()
        cp2.wait()
        cp3.wait()
        cp4.wait()
        plsc.subcore_barrier()

        def blk1(j, carry):
            for r in range(B // L):
                si = srcv[j, pl.ds(r * L, L)]
                di = dstv[j, pl.ds(r * L, L)]
                a = plsc.load_gather(s1v, [si]) + plsc.load_gather(s2v, [di])
                a = jnp.where(a >= 0, a, SLOPE * a)
                alphav[j, pl.ds(r * L, L)] = jnp.exp(a)
            pltpu.sync_copy(alphav.at[j], den_sh.at[srcv.at[j]], add=True)
            return carry
        lax.fori_loop(0, nblk, blk1, 0)

        def blk2(j, carry):
            pltpu.async_copy(wi_hbm.at[dstv.at[j]], rows, sem).wait()

            def row_body(b, carry2):
                ab = plsc.load_gather(
                    alphav,
                    [jnp.full((L,), j, jnp.int32), jnp.full((L,), b, jnp.int32)])
                for k in range(d // L):
                    rows[b, pl.ds(k * L, L)] = rows[b, pl.ds(k * L, L)] * ab
                return carry2
            lax.fori_loop(0, B, row_body, 0)
            pltpu.sync_copy(rows, out_sh.at[srcv.at[j]], add=True)
            return carry
        lax.fori_loop(0, nblk, blk2, 0)

        plsc.subcore_barrier()

        for k in range(rpt // B):
            pltpu.sync_copy(out_sh.at[pl.ds(r0 + k * B, B)],
                            outp_hbm.at[c, pl.ds(r0 + k * B, B)])
        pltpu.sync_copy(den_sh.at[pl.ds(r0, rpt)],
                        denp_hbm.at[c, pl.ds(r0, rpt)])

    return edge_kernel


@jax.jit
def kernel(input_matrix, adjacency_coo_matrix, weights_matrix,
           attention_bias_vector):
    n, d = input_matrix.shape
    e = adjacency_coo_matrix.shape[1]

    npad = ((n + NS * B - 1) // (NS * B)) * (NS * B)
    ep = e + n
    nblk = (ep + NW * B - 1) // (NW * B)
    etot = NW * nblk * B

    x_pad = jnp.zeros((npad, d), jnp.float32).at[:n].set(input_matrix)
    b2p = jnp.zeros((d, 8), jnp.float32)
    b2p = b2p.at[:, 0].set(attention_bias_vector[:d])
    b2p = b2p.at[:, 1].set(attention_bias_vector[d:])

    wi, s_out = pl.pallas_call(
        _matmul_scores_kernel,
        grid=(npad // B,),
        in_specs=[
            pl.BlockSpec((B, d), lambda i: (i, 0)),
            pl.BlockSpec((d, d), lambda i: (0, 0)),
            pl.BlockSpec((d, 8), lambda i: (0, 0)),
        ],
        out_specs=[
            pl.BlockSpec((B, d), lambda i: (i, 0)),
            pl.BlockSpec((8, B), lambda i: (0, i)),
        ],
        out_shape=[
            jax.ShapeDtypeStruct((npad, d), jnp.float32),
            jax.ShapeDtypeStruct((8, npad), jnp.float32),
        ],
    )(x_pad, weights_matrix, b2p)
    s1 = s_out[0]
    s2 = s_out[1]

    self_idx = jnp.arange(n, dtype=jnp.int32)
    src = jnp.concatenate([
        adjacency_coo_matrix[0], self_idx,
        jnp.full((etot - ep,), npad - 1, jnp.int32)])
    dst = jnp.concatenate([
        adjacency_coo_matrix[1], self_idx,
        jnp.zeros((etot - ep,), jnp.int32)])
    src3 = src.reshape(NW, nblk, B)
    dst3 = dst.reshape(NW, nblk, B)

    outp, denp = _make_edge_kernel(npad, d, nblk)(wi, s1, s2, src3, dst3)

    out_pad = pl.pallas_call(
        _combine_kernel,
        grid=(npad // B,),
        in_specs=[
            pl.BlockSpec((NC, B, d), lambda i: (0, i, 0)),
            pl.BlockSpec((NC, B, 1), lambda i: (0, i, 0)),
        ],
        out_specs=pl.BlockSpec((B, d), lambda i: (i, 0)),
        out_shape=jax.ShapeDtypeStruct((npad, d), jnp.float32),
    )(outp, denp.reshape(NC, npad, 1))

    return out_pad[:n]


# speedup vs baseline: 11.4857x; 11.4857x over previous
import functools

import jax
import jax.numpy as jnp
from jax import lax
from jax.experimental import pallas as pl
from jax.experimental.pallas import tpu as pltpu
from jax.experimental.pallas import tpu_sc as plsc

SLOPE = 0.2
L = 16
NC = 2
NS = 16
NW = NC * NS
B = 128


def _matmul_scores_kernel(x_ref, w_ref, b_ref, wi_ref, s_ref):
    wi = jnp.dot(x_ref[...], w_ref[...], preferred_element_type=jnp.float32)
    wi_ref[...] = wi
    s_ref[...] = lax.dot_general(
        b_ref[...], wi, (((0,), (1,)), ((), ())),
        preferred_element_type=jnp.float32)


def _combine_kernel(p_ref, d_ref, o_ref):
    o_ref[...] = (p_ref[0] + p_ref[1]) / (d_ref[0] + d_ref[1])


def _quota(nout):
    return ((nout + NS - 1) // NS + 7) // 8 * 8


def _sc_mesh():
    return plsc.VectorSubcoreMesh(
        core_axis_name="c", subcore_axis_name="s",
        num_cores=NC, num_subcores=NS)


def _make_alpha_kernel(nout, nsc, nblk):
    q = _quota(nout)

    @functools.partial(
        pl.kernel,
        out_type=(
            jax.ShapeDtypeStruct((NW, nblk, B), jnp.float32),
            jax.ShapeDtypeStruct((NC * nout,), jnp.float32),
        ),
        mesh=_sc_mesh(),
        scratch_types=[
            pltpu.VMEM((nsc // 128, 128), jnp.float32),
            pltpu.VMEM((nsc // 128, 128), jnp.float32),
            pltpu.VMEM((nblk, B), jnp.int32),
            pltpu.VMEM((nblk, B), jnp.int32),
            pltpu.VMEM((nblk, B), jnp.float32),
            pltpu.VMEM((q,), jnp.float32),
            pltpu.VMEM_SHARED((nout,), jnp.float32),
            pltpu.SemaphoreType.DMA,
            pltpu.SemaphoreType.DMA,
        ],
        compiler_params=pltpu.CompilerParams(needs_layout_passes=False),
    )
    def alpha_kernel(s1_hbm, s2_hbm, src_hbm, dst_hbm,
                     alpha_hbm, denp_hbm,
                     s1v, s2v, srcv, dstv, alphav, zden, den_sh, sem, semd):
        c = lax.axis_index("c")
        s = lax.axis_index("s")
        w = s * NC + c
        r0 = jnp.minimum(s * q, nout - q)

        cp1 = pltpu.async_copy(s1_hbm, s1v, sem)
        cp2 = pltpu.async_copy(s2_hbm, s2v, sem)
        cp3 = pltpu.async_copy(src_hbm.at[w], srcv, sem)
        cp4 = pltpu.async_copy(dst_hbm.at[w], dstv, sem)

        for k in range(q // L):
            zden[pl.ds(k * L, L)] = jnp.zeros((L,), jnp.float32)
        pltpu.sync_copy(zden, den_sh.at[pl.ds(r0, q)])

        cp1.wait()
        cp2.wait()
        cp3.wait()
        cp4.wait()
        plsc.subcore_barrier()

        def blk1(j, carry):
            for r in range(B // L):
                si = srcv[j, pl.ds(r * L, L)]
                di = dstv[j, pl.ds(r * L, L)]
                a = (plsc.load_gather(s1v, [si >> 7, si & 127])
                     + plsc.load_gather(s2v, [di >> 7, di & 127]))
                a = jnp.where(a >= 0, a, SLOPE * a)
                alphav[j, pl.ds(r * L, L)] = jnp.exp(a)
            pltpu.async_copy(alphav.at[j], den_sh.at[srcv.at[j]], semd,
                             add=True)
            return carry
        lax.fori_loop(0, nblk, blk1, 0)

        pltpu.sync_copy(alphav, alpha_hbm.at[w])

        def drain(j, carry):
            pltpu.make_async_copy(alphav.at[j], den_sh.at[srcv.at[j]],
                                  semd).wait()
            return carry
        lax.fori_loop(0, nblk, drain, 0)
        plsc.subcore_barrier()

        pltpu.sync_copy(den_sh.at[pl.ds(r0, q)], zden)
        pltpu.sync_copy(zden, denp_hbm.at[pl.ds(c * nout + r0, q)])

    return alpha_kernel


def _make_rows_kernel(nout, d, nblk):
    q = _quota(nout)

    @functools.partial(
        pl.kernel,
        out_type=jax.ShapeDtypeStruct((NC, nout, d), jnp.float32),
        mesh=_sc_mesh(),
        scratch_types=[
            pltpu.VMEM((nblk, B), jnp.int32),
            pltpu.VMEM((nblk, B), jnp.int32),
            pltpu.VMEM((nblk, B), jnp.float32),
            pltpu.VMEM((B, d), jnp.float32),
            pltpu.VMEM_SHARED((nout, d), jnp.float32),
            pltpu.SemaphoreType.DMA,
        ],
        compiler_params=pltpu.CompilerParams(needs_layout_passes=False),
    )
    def rows_kernel(wi_hbm, src_hbm, dst_hbm, alpha_hbm, outp_hbm,
                    srcv, dstv, alphav, rows, out_sh, sem):
        c = lax.axis_index("c")
        s = lax.axis_index("s")
        w = s * NC + c
        r0 = jnp.minimum(s * q, nout - q)

        cp3 = pltpu.async_copy(src_hbm.at[w], srcv, sem)
        cp4 = pltpu.async_copy(dst_hbm.at[w], dstv, sem)
        cp5 = pltpu.async_copy(alpha_hbm.at[w], alphav, sem)

        def zrow_body(b, carry):
            for k in range(d // L):
                rows[b, pl.ds(k * L, L)] = jnp.zeros((L,), jnp.float32)
            return carry
        lax.fori_loop(0, B, zrow_body, 0)

        off = 0
        left = q
        while left > 0:
            csz = min(B, left)
            pltpu.sync_copy(rows.at[pl.ds(0, csz)],
                            out_sh.at[pl.ds(r0 + off, csz)])
            off += csz
            left -= csz

        cp3.wait()
        cp4.wait()
        cp5.wait()
        plsc.subcore_barrier()

        def blk2(j, carry):
            pltpu.async_copy(wi_hbm.at[dstv.at[j]], rows, sem).wait()
            jv = jnp.full((L,), j, jnp.int32)

            def row_body(b2, carry2):
                b = b2 * 2
                ab0 = plsc.load_gather(
                    alphav, [jv, jnp.full((L,), b, jnp.int32)])
                ab1 = plsc.load_gather(
                    alphav, [jv, jnp.full((L,), b + 1, jnp.int32)])
                for k in range(d // L):
                    rows[b, pl.ds(k * L, L)] = rows[b, pl.ds(k * L, L)] * ab0
                for k in range(d // L):
                    rows[b + 1, pl.ds(k * L, L)] = (
                        rows[b + 1, pl.ds(k * L, L)] * ab1)
                return carry2
            lax.fori_loop(0, B // 2, row_body, 0)
            pltpu.sync_copy(rows, out_sh.at[srcv.at[j]], add=True)
            return carry
        lax.fori_loop(0, nblk, blk2, 0)

        plsc.subcore_barrier()
        pltpu.sync_copy(out_sh.at[pl.ds(r0, q)],
                        outp_hbm.at[c, pl.ds(r0, q)])

    return rows_kernel


@jax.jit
def kernel(input_matrix, adjacency_coo_matrix, weights_matrix,
           attention_bias_vector):
    n, d = input_matrix.shape
    e = adjacency_coo_matrix.shape[1]

    nout = (n + 1 + 7) // 8 * 8
    trash = nout - 1
    nsc = (nout + 127) // 128 * 128
    ep = e + n
    nblk = (ep + NW * B - 1) // (NW * B)
    etot = NW * nblk * B

    x_pad = jnp.zeros((nsc, d), jnp.float32).at[:n].set(input_matrix)
    b2p = jnp.zeros((d, 8), jnp.float32)
    b2p = b2p.at[:, 0].set(attention_bias_vector[:d])
    b2p = b2p.at[:, 1].set(attention_bias_vector[d:])

    wi, s_out = pl.pallas_call(
        _matmul_scores_kernel,
        grid=(nsc // B,),
        in_specs=[
            pl.BlockSpec((B, d), lambda i: (i, 0)),
            pl.BlockSpec((d, d), lambda i: (0, 0)),
            pl.BlockSpec((d, 8), lambda i: (0, 0)),
        ],
        out_specs=[
            pl.BlockSpec((B, d), lambda i: (i, 0)),
            pl.BlockSpec((8, B), lambda i: (0, i)),
        ],
        out_shape=[
            jax.ShapeDtypeStruct((nsc, d), jnp.float32),
            jax.ShapeDtypeStruct((8, nsc), jnp.float32),
        ],
    )(x_pad, weights_matrix, b2p)
    s1 = s_out[0].reshape(nsc // 128, 128)
    s2 = s_out[1].reshape(nsc // 128, 128)

    self_idx = jnp.arange(n, dtype=jnp.int32)
    src = jnp.concatenate([
        adjacency_coo_matrix[0], self_idx,
        jnp.full((etot - ep,), trash, jnp.int32)])
    dst = jnp.concatenate([
        adjacency_coo_matrix[1], self_idx,
        jnp.zeros((etot - ep,), jnp.int32)])
    src3 = src.reshape(NW, nblk, B)
    dst3 = dst.reshape(NW, nblk, B)

    alpha, denp = _make_alpha_kernel(nout, nsc, nblk)(s1, s2, src3, dst3)

    outp = _make_rows_kernel(nout, d, nblk)(wi, src3, dst3, alpha)

    bs = 8
    for cand in range(256, 7, -8):
        if nout % cand == 0:
            bs = cand
            break
    out_pad = pl.pallas_call(
        _combine_kernel,
        grid=(nout // bs,),
        in_specs=[
            pl.BlockSpec((NC, bs, d), lambda i: (0, i, 0)),
            pl.BlockSpec((NC, bs, 1), lambda i: (0, i, 0)),
        ],
        out_specs=pl.BlockSpec((bs, d), lambda i: (i, 0)),
        out_shape=jax.ShapeDtypeStruct((nout, d), jnp.float32),
    )(outp, denp.reshape(NC, nout, 1))

    return out_pad[:n]
